# Initial kernel scaffold; baseline (speedup 1.0000x reference)
import jax
import jax.numpy as jnp
from jax.experimental import pallas as pl


def kernel(x, edge_index, edge_attr, batch, Wm_w, Wm_b, Wa_w, Wa_b, bn_gamma, bn_beta, P1_w, P1_b, P2_w, P2_b):
    raise NotImplementedError("write your pallas kernel here")


# speedup vs baseline: 1.5349x; 1.5349x over previous
import jax
import jax.numpy as jnp
from jax.experimental import pallas as pl
from jax.experimental.pallas import tpu as pltpu

G = 64
EPS = 1e-5
F32 = jnp.float32


def _e1_body(col_ref, ea_ref, acc_ref, vals_ref):
    i = pl.program_id(0)

    @pl.when(i == 0)
    def _():
        acc_ref[...] = jnp.zeros_like(acc_ref)

    eb = ea_ref.shape[0]
    de = ea_ref.shape[1]
    vals_ref[...] = jnp.concatenate(
        [ea_ref[...], jnp.ones((eb, 1), F32), jnp.zeros((eb, 127 - de), F32)],
        axis=1)

    def body(j, carry):
        c = col_ref[0, 0, j]
        acc_ref[pl.ds(c, 1), :] += vals_ref[pl.ds(j, 1), :]
        return carry

    jax.lax.fori_loop(0, eb, body, 0)


def _d1_body(x_ref, acc1_ref, wmx_ref, wme_ref, wmb_ref,
             xwb_ref, normb_ref, selfc_ref):
    de = wme_ref.shape[0]
    cnt = acc1_ref[:, de:de + 1]
    loop_sum = acc1_ref[:, :de]
    deg = cnt + 1.0
    norm = jax.lax.rsqrt(deg)
    la = loop_sum / jnp.maximum(cnt, 1.0)
    xwb = (jnp.dot(x_ref[...], wmx_ref[...], preferred_element_type=F32)
           + wmb_ref[...])
    selfc = jnp.maximum(
        xwb + jnp.dot(la, wme_ref[...], preferred_element_type=F32), 0.0) / deg
    xwb_ref[...] = xwb
    normb_ref[...] = jnp.broadcast_to(norm, normb_ref.shape)
    selfc_ref[...] = selfc


def _e2_body(row_ref, col_ref, ea_ref, wme_ref, xwb_ref, normb_ref,
             acc_ref, pre_ref):
    i = pl.program_id(0)

    @pl.when(i == 0)
    def _():
        acc_ref[...] = jnp.zeros_like(acc_ref)

    eb = ea_ref.shape[0]
    pre_ref[...] = jnp.dot(ea_ref[...], wme_ref[...], preferred_element_type=F32)

    def body(j, carry):
        r = row_ref[0, 0, j]
        c = col_ref[0, 0, j]
        xrow = xwb_ref[pl.ds(r, 1), :]
        m = jnp.maximum(xrow + pre_ref[pl.ds(j, 1), :], 0.0) \
            * normb_ref[pl.ds(r, 1), :]
        acc_ref[pl.ds(c, 1), :] += m
        return carry

    jax.lax.fori_loop(0, eb, body, 0)


def _d2_body(x_ref, acc2_ref, selfc_ref, normb_ref, batch_ref,
             wax_ref, waa_ref, wab_ref, sums_ref, cnts_ref):
    i = pl.program_id(0)

    @pl.when(i == 0)
    def _():
        sums_ref[...] = jnp.zeros_like(sums_ref)
        cnts_ref[...] = jnp.zeros_like(cnts_ref)

    aggr = acc2_ref[...] * normb_ref[...] + selfc_ref[...]
    h = jnp.maximum(
        jnp.dot(x_ref[...], wax_ref[...], preferred_element_type=F32)
        + jnp.dot(aggr, waa_ref[...], preferred_element_type=F32)
        + wab_ref[...], 0.0)
    oh = (batch_ref[...] ==
          jax.lax.broadcasted_iota(jnp.int32, (1, G), 1)).astype(F32)
    sums_ref[...] += jax.lax.dot_general(
        oh, h, (((0,), (0,)), ((), ())), preferred_element_type=F32)
    cnts_ref[...] += jnp.broadcast_to(
        jnp.sum(oh, axis=0)[:, None], cnts_ref.shape)


def _d3_body(sums_ref, cnts_ref, bng_ref, bnb_ref, p1t_ref, p1b_ref,
             p2t_ref, p2b_ref, out_ref):
    cnt = jnp.maximum(cnts_ref[:, :1], 1.0)
    g = sums_ref[...] / cnt
    g = g * (bng_ref[...] / jnp.sqrt(1.0 + EPS)) + bnb_ref[...]
    r = jnp.maximum(
        jnp.dot(g, p1t_ref[...], preferred_element_type=F32) + p1b_ref[...],
        0.0)
    out_ref[...] = (jnp.dot(r, p2t_ref[...], preferred_element_type=F32)
                    + p2b_ref[...])


def kernel(x, edge_index, edge_attr, batch, Wm_w, Wm_b, Wa_w, Wa_b,
           bn_gamma, bn_beta, P1_w, P1_b, P2_w, P2_b):
    n, d = x.shape
    e = edge_index.shape[1]
    de = edge_attr.shape[1]
    h = Wa_w.shape[0]
    p = P1_w.shape[0]
    t = P2_w.shape[0]

    eb = 1000 if e % 1000 == 0 else e
    nb = 1000 if n % 1000 == 0 else n

    row3 = edge_index[0].reshape(e // eb, 1, eb)
    col3 = edge_index[1].reshape(e // eb, 1, eb)
    wmxT = Wm_w[:, :d].T
    wmeT = Wm_w[:, d:].T
    wmb2 = Wm_b.reshape(1, d)
    waxT = Wa_w[:, :d].T
    waaT = Wa_w[:, d:].T
    wab2 = Wa_b.reshape(1, h)
    batch2 = batch.reshape(n, 1)

    acc1 = pl.pallas_call(
        _e1_body,
        grid=(e // eb,),
        in_specs=[
            pl.BlockSpec((1, 1, eb), lambda i: (i, 0, 0),
                         memory_space=pltpu.SMEM),
            pl.BlockSpec((eb, de), lambda i: (i, 0)),
        ],
        out_specs=pl.BlockSpec((n, 128), lambda i: (0, 0)),
        out_shape=jax.ShapeDtypeStruct((n, 128), F32),
        scratch_shapes=[pltpu.VMEM((eb, 128), F32)],
    )(col3, edge_attr)

    xwb, normb, selfc = pl.pallas_call(
        _d1_body,
        grid=(n // nb,),
        in_specs=[
            pl.BlockSpec((nb, d), lambda i: (i, 0)),
            pl.BlockSpec((nb, 128), lambda i: (i, 0)),
            pl.BlockSpec((d, d), lambda i: (0, 0)),
            pl.BlockSpec((de, d), lambda i: (0, 0)),
            pl.BlockSpec((1, d), lambda i: (0, 0)),
        ],
        out_specs=[
            pl.BlockSpec((nb, d), lambda i: (i, 0)),
            pl.BlockSpec((nb, 128), lambda i: (i, 0)),
            pl.BlockSpec((nb, d), lambda i: (i, 0)),
        ],
        out_shape=[
            jax.ShapeDtypeStruct((n, d), F32),
            jax.ShapeDtypeStruct((n, 128), F32),
            jax.ShapeDtypeStruct((n, d), F32),
        ],
    )(x, acc1, wmxT, wmeT, wmb2)

    acc2 = pl.pallas_call(
        _e2_body,
        grid=(e // eb,),
        in_specs=[
            pl.BlockSpec((1, 1, eb), lambda i: (i, 0, 0),
                         memory_space=pltpu.SMEM),
            pl.BlockSpec((1, 1, eb), lambda i: (i, 0, 0),
                         memory_space=pltpu.SMEM),
            pl.BlockSpec((eb, de), lambda i: (i, 0)),
            pl.BlockSpec((de, d), lambda i: (0, 0)),
            pl.BlockSpec((n, d), lambda i: (0, 0)),
            pl.BlockSpec((n, 128), lambda i: (0, 0)),
        ],
        out_specs=pl.BlockSpec((n, d), lambda i: (0, 0)),
        out_shape=jax.ShapeDtypeStruct((n, d), F32),
        scratch_shapes=[pltpu.VMEM((eb, d), F32)],
    )(row3, col3, edge_attr, wmeT, xwb, normb)

    sums, cnts = pl.pallas_call(
        _d2_body,
        grid=(n // nb,),
        in_specs=[
            pl.BlockSpec((nb, d), lambda i: (i, 0)),
            pl.BlockSpec((nb, d), lambda i: (i, 0)),
            pl.BlockSpec((nb, d), lambda i: (i, 0)),
            pl.BlockSpec((nb, 128), lambda i: (i, 0)),
            pl.BlockSpec((nb, 1), lambda i: (i, 0)),
            pl.BlockSpec((d, h), lambda i: (0, 0)),
            pl.BlockSpec((d, h), lambda i: (0, 0)),
            pl.BlockSpec((1, h), lambda i: (0, 0)),
        ],
        out_specs=[
            pl.BlockSpec((G, h), lambda i: (0, 0)),
            pl.BlockSpec((G, 128), lambda i: (0, 0)),
        ],
        out_shape=[
            jax.ShapeDtypeStruct((G, h), F32),
            jax.ShapeDtypeStruct((G, 128), F32),
        ],
    )(x, acc2, selfc, normb, batch2, waxT, waaT, wab2)

    out = pl.pallas_call(
        _d3_body,
        in_specs=[
            pl.BlockSpec((G, h), lambda: (0, 0)),
            pl.BlockSpec((G, 128), lambda: (0, 0)),
            pl.BlockSpec((1, h), lambda: (0, 0)),
            pl.BlockSpec((1, h), lambda: (0, 0)),
            pl.BlockSpec((h, p), lambda: (0, 0)),
            pl.BlockSpec((1, p), lambda: (0, 0)),
            pl.BlockSpec((p, t), lambda: (0, 0)),
            pl.BlockSpec((1, t), lambda: (0, 0)),
        ],
        out_specs=pl.BlockSpec((G, t), lambda: (0, 0)),
        out_shape=jax.ShapeDtypeStruct((G, t), F32),
    )(sums, cnts, bn_gamma.reshape(1, h), bn_beta.reshape(1, h),
      P1_w.T, P1_b.reshape(1, p), P2_w.T, P2_b.reshape(1, t))

    return out


# speedup vs baseline: 6.5524x; 4.2691x over previous
import functools

import jax
import jax.numpy as jnp
from jax import lax
from jax.experimental import pallas as pl
from jax.experimental.pallas import tpu as pltpu
from jax.experimental.pallas import tpu_sc as plsc

G = 64
EPS = 1e-5
F32 = jnp.float32

NC = 2
NS = 16
NW = NC * NS
L = 16
K = 80
NZ = 10


def _sc_mesh():
    return plsc.VectorSubcoreMesh(core_axis_name="c", subcore_axis_name="s")


def _p1_body(n, col_hbm, ea_hbm, z128_hbm, out_hbm,
             cidx_v, ea_v, w_v, acc_sh, sem):
    c = lax.axis_index("c")
    s = lax.axis_index("s")
    wid = s * NC + c
    de = ea_v.shape[1]
    rps = n // NZ

    one16 = jnp.where(lax.iota(jnp.int32, L) == 0, 1.0, 0.0).astype(F32)
    zero16 = jnp.zeros((L,), F32)

    def init_row(r, _):
        w_v[r, pl.ds(de, L)] = one16
        for cc in range(2 * de, 128, L):
            w_v[r, pl.ds(cc, L)] = zero16
        return 0

    lax.fori_loop(0, K, init_row, 0)

    @pl.when(s < NZ)
    def _():
        pltpu.sync_copy(z128_hbm.at[pl.ds(s * rps, rps)],
                        acc_sh.at[pl.ds(s * rps, rps)])
    plsc.subcore_barrier()

    ngr, ngc = col_hbm.shape[1], col_hbm.shape[2]
    ebase = wid * (ngr * ngc * K)

    def group(g, _):
        pltpu.sync_copy(col_hbm.at[wid, g], cidx_v)

        def chunk(ch, _):
            base = ebase + (g * ngc + ch) * K
            pltpu.sync_copy(ea_hbm.at[pl.ds(base, K)], ea_v)

            def row(r, _):
                w_v[r, pl.ds(0, de)] = ea_v[r, pl.ds(0, de)]
                return 0

            lax.fori_loop(0, K, row, 0)
            pltpu.sync_copy(w_v, acc_sh.at[cidx_v.at[ch]], add=True)
            return 0

        lax.fori_loop(0, ngc, chunk, 0)
        return 0

    lax.fori_loop(0, ngr, group, 0)
    plsc.subcore_barrier()

    @pl.when(s < NZ)
    def _():
        pltpu.sync_copy(acc_sh.at[pl.ds(s * rps, rps)],
                        out_hbm.at[pl.ds(c * n + s * rps, rps)])


def _p5_body(n, d, row_hbm, col_hbm, ew_hbm, gt_hbm, z128_hbm,
             out_hbm, ridx_v, cidx_v, ebuf, gbuf, acc_sh, sem):
    c = lax.axis_index("c")
    s = lax.axis_index("s")
    wid = s * NC + c
    rps = n // NZ

    @pl.when(s < NZ)
    def _():
        pltpu.sync_copy(z128_hbm.at[pl.ds(s * rps, rps)],
                        acc_sh.at[pl.ds(s * rps, rps)])
    plsc.subcore_barrier()

    ngr, ngc = col_hbm.shape[1], col_hbm.shape[2]
    ebase = wid * (ngr * ngc * K)

    def group(g, _):
        pltpu.sync_copy(row_hbm.at[wid, g], ridx_v)
        pltpu.sync_copy(col_hbm.at[wid, g], cidx_v)

        def chunk(ch, _):
            base = ebase + (g * ngc + ch) * K
            pltpu.sync_copy(ew_hbm.at[pl.ds(base, K)], ebuf)
            pltpu.async_copy(gt_hbm.at[ridx_v.at[ch]], gbuf, sem).wait()

            def row(r, _):
                nr = gbuf[r, pl.ds(d, L)]
                for cc in range(d // L):
                    sl = pl.ds(cc * L, L)
                    ebuf[r, sl] = jnp.maximum(
                        gbuf[r, sl] + ebuf[r, sl] * nr, 0.0)
                return 0

            lax.fori_loop(0, K, row, 0)
            pltpu.sync_copy(ebuf, acc_sh.at[cidx_v.at[ch]], add=True)
            return 0

        lax.fori_loop(0, ngc, chunk, 0)
        return 0

    lax.fori_loop(0, ngr, group, 0)
    plsc.subcore_barrier()

    @pl.when(s < NZ)
    def _():
        pltpu.sync_copy(acc_sh.at[pl.ds(s * rps, rps)],
                        out_hbm.at[pl.ds(c * n + s * rps, rps)])


def _p2_body(x_ref, p0_ref, p1_ref, wmx_ref, wme_ref, wmb_ref,
             gt_ref, selfc_ref, normb_ref):
    de = wme_ref.shape[0]
    cnt = p0_ref[:, de:de + 1] + p1_ref[:, de:de + 1]
    loop_sum = p0_ref[:, :de] + p1_ref[:, :de]
    deg = cnt + 1.0
    norm = lax.rsqrt(deg)
    la = loop_sum / jnp.maximum(cnt, 1.0)
    xwb = (jnp.dot(x_ref[...], wmx_ref[...], preferred_element_type=F32)
           + wmb_ref[...])
    selfc = jnp.maximum(
        xwb + jnp.dot(la, wme_ref[...], preferred_element_type=F32), 0.0) / deg
    nb = jnp.broadcast_to(norm, normb_ref.shape)
    gt_ref[...] = jnp.concatenate([xwb * norm, nb], axis=1)
    selfc_ref[...] = selfc
    normb_ref[...] = nb


def _p4_body(ea_ref, wme_ref, out_ref):
    out_ref[...] = jnp.dot(ea_ref[...], wme_ref[...],
                           preferred_element_type=F32)


def _p6_body(x_ref, a0_ref, a1_ref, selfc_ref, normb_ref, batch_ref,
             wax_ref, waa_ref, wab_ref, sums_ref, cnts_ref):
    i = pl.program_id(0)

    @pl.when(i == 0)
    def _():
        sums_ref[...] = jnp.zeros_like(sums_ref)
        cnts_ref[...] = jnp.zeros_like(cnts_ref)

    aggr = (a0_ref[...] + a1_ref[...]) * normb_ref[...] + selfc_ref[...]
    h = jnp.maximum(
        jnp.dot(x_ref[...], wax_ref[...], preferred_element_type=F32)
        + jnp.dot(aggr, waa_ref[...], preferred_element_type=F32)
        + wab_ref[...], 0.0)
    oh = (batch_ref[...] ==
          lax.broadcasted_iota(jnp.int32, (1, G), 1)).astype(F32)
    sums_ref[...] += lax.dot_general(
        oh, h, (((0,), (0,)), ((), ())), preferred_element_type=F32)
    cnts_ref[...] += jnp.broadcast_to(
        jnp.sum(oh, axis=0)[:, None], cnts_ref.shape)


def _d3_body(sums_ref, cnts_ref, bng_ref, bnb_ref, p1t_ref, p1b_ref,
             p2t_ref, p2b_ref, out_ref):
    cnt = jnp.maximum(cnts_ref[:, :1], 1.0)
    g = sums_ref[...] / cnt
    g = g * (bng_ref[...] / jnp.sqrt(1.0 + EPS)) + bnb_ref[...]
    r = jnp.maximum(
        jnp.dot(g, p1t_ref[...], preferred_element_type=F32) + p1b_ref[...],
        0.0)
    out_ref[...] = (jnp.dot(r, p2t_ref[...], preferred_element_type=F32)
                    + p2b_ref[...])


def kernel(x, edge_index, edge_attr, batch, Wm_w, Wm_b, Wa_w, Wa_b,
           bn_gamma, bn_beta, P1_w, P1_b, P2_w, P2_b):
    n, d = x.shape
    e = edge_index.shape[1]
    de = edge_attr.shape[1]
    h = Wa_w.shape[0]
    p = P1_w.shape[0]
    t = P2_w.shape[0]

    et = e // NW
    nch = et // K
    ngc = 25 if nch % 25 == 0 else nch
    ngr = nch // ngc
    nb = 1000 if n % 1000 == 0 else n
    eb = 2000 if e % 2000 == 0 else e

    row3 = edge_index[0].reshape(NW, ngr, ngc, K)
    col3 = edge_index[1].reshape(NW, ngr, ngc, K)
    wmxT = Wm_w[:, :d].T
    wmeT = Wm_w[:, d:].T
    wmb2 = Wm_b.reshape(1, d)
    waxT = Wa_w[:, :d].T
    waaT = Wa_w[:, d:].T
    wab2 = Wa_b.reshape(1, h)
    batch2 = batch.reshape(n, 1)
    z128 = jnp.zeros((n, 128), F32)

    mesh = _sc_mesh()

    parts1 = pl.kernel(
        functools.partial(_p1_body, n),
        out_type=jax.ShapeDtypeStruct((NC * n, 128), F32),
        mesh=mesh,
        scratch_types=[
            pltpu.VMEM((ngc, K), jnp.int32),
            pltpu.VMEM((K, de), F32),
            pltpu.VMEM((K, 128), F32),
            pltpu.VMEM_SHARED((n, 128), F32),
            pltpu.SemaphoreType.DMA,
        ],
    )(col3, edge_attr, z128)

    gt, selfc, normb = pl.pallas_call(
        _p2_body,
        grid=(n // nb,),
        in_specs=[
            pl.BlockSpec((nb, d), lambda i: (i, 0)),
            pl.BlockSpec((nb, 128), lambda i: (i, 0)),
            pl.BlockSpec((nb, 128), lambda i, _m=n // nb: (i + _m, 0)),
            pl.BlockSpec((d, d), lambda i: (0, 0)),
            pl.BlockSpec((de, d), lambda i: (0, 0)),
            pl.BlockSpec((1, d), lambda i: (0, 0)),
        ],
        out_specs=[
            pl.BlockSpec((nb, d + 128), lambda i: (i, 0)),
            pl.BlockSpec((nb, d), lambda i: (i, 0)),
            pl.BlockSpec((nb, 128), lambda i: (i, 0)),
        ],
        out_shape=[
            jax.ShapeDtypeStruct((n, d + 128), F32),
            jax.ShapeDtypeStruct((n, d), F32),
            jax.ShapeDtypeStruct((n, 128), F32),
        ],
    )(x, parts1, parts1, wmxT, wmeT, wmb2)

    ew = pl.pallas_call(
        _p4_body,
        grid=(e // eb,),
        in_specs=[
            pl.BlockSpec((eb, de), lambda i: (i, 0)),
            pl.BlockSpec((de, d), lambda i: (0, 0)),
        ],
        out_specs=pl.BlockSpec((eb, d), lambda i: (i, 0)),
        out_shape=jax.ShapeDtypeStruct((e, d), F32),
    )(edge_attr, wmeT)

    aggr_parts = pl.kernel(
        functools.partial(_p5_body, n, d),
        out_type=jax.ShapeDtypeStruct((NC * n, d), F32),
        mesh=mesh,
        scratch_types=[
            pltpu.VMEM((ngc, K), jnp.int32),
            pltpu.VMEM((ngc, K), jnp.int32),
            pltpu.VMEM((K, d), F32),
            pltpu.VMEM((K, d + 128), F32),
            pltpu.VMEM_SHARED((n, d), F32),
            pltpu.SemaphoreType.DMA,
        ],
    )(row3, col3, ew, gt, z128)

    sums, cnts = pl.pallas_call(
        _p6_body,
        grid=(n // nb,),
        in_specs=[
            pl.BlockSpec((nb, d), lambda i: (i, 0)),
            pl.BlockSpec((nb, d), lambda i: (i, 0)),
            pl.BlockSpec((nb, d), lambda i, _m=n // nb: (i + _m, 0)),
            pl.BlockSpec((nb, d), lambda i: (i, 0)),
            pl.BlockSpec((nb, 128), lambda i: (i, 0)),
            pl.BlockSpec((nb, 1), lambda i: (i, 0)),
            pl.BlockSpec((d, h), lambda i: (0, 0)),
            pl.BlockSpec((d, h), lambda i: (0, 0)),
            pl.BlockSpec((1, h), lambda i: (0, 0)),
        ],
        out_specs=[
            pl.BlockSpec((G, h), lambda i: (0, 0)),
            pl.BlockSpec((G, 128), lambda i: (0, 0)),
        ],
        out_shape=[
            jax.ShapeDtypeStruct((G, h), F32),
            jax.ShapeDtypeStruct((G, 128), F32),
        ],
    )(x, aggr_parts, aggr_parts, selfc, normb, batch2, waxT, waaT, wab2)

    out = pl.pallas_call(
        _d3_body,
        in_specs=[
            pl.BlockSpec((G, h), lambda: (0, 0)),
            pl.BlockSpec((G, 128), lambda: (0, 0)),
            pl.BlockSpec((1, h), lambda: (0, 0)),
            pl.BlockSpec((1, h), lambda: (0, 0)),
            pl.BlockSpec((h, p), lambda: (0, 0)),
            pl.BlockSpec((1, p), lambda: (0, 0)),
            pl.BlockSpec((p, t), lambda: (0, 0)),
            pl.BlockSpec((1, t), lambda: (0, 0)),
        ],
        out_specs=pl.BlockSpec((G, t), lambda: (0, 0)),
        out_shape=jax.ShapeDtypeStruct((G, t), F32),
    )(sums, cnts, bn_gamma.reshape(1, h), bn_beta.reshape(1, h),
      P1_w.T, P1_b.reshape(1, p), P2_w.T, P2_b.reshape(1, t))

    return out


# speedup vs baseline: 12.8767x; 1.9652x over previous
import functools

import jax
import jax.numpy as jnp
from jax import lax
from jax.experimental import pallas as pl
from jax.experimental.pallas import tpu as pltpu
from jax.experimental.pallas import tpu_sc as plsc

G = 64
EPS = 1e-5
F32 = jnp.float32

NC = 2
NS = 16
NW = NC * NS
L = 16
K = 80
NZ = 10


def _sc_mesh():
    return plsc.VectorSubcoreMesh(core_axis_name="c", subcore_axis_name="s")


def _p1_body(n, kk, col_hbm, ea_hbm, z128_hbm, out_hbm,
             cidx_v, ea_v, w_v, acc_sh, sem_e, sem_s):
    c = lax.axis_index("c")
    s = lax.axis_index("s")
    wid = s * NC + c
    de = ea_v.shape[2]
    rps = n // NZ

    one16 = jnp.where(lax.iota(jnp.int32, L) == 0, 1.0, 0.0).astype(F32)
    zero16 = jnp.zeros((L,), F32)

    for ws in range(w_v.shape[0]):
        @plsc.parallel_loop(0, kk, unroll=8)
        def _(r, _ws=ws):
            w_v[_ws, r, pl.ds(de, L)] = one16
            for cc in range(2 * de, 128, L):
                w_v[_ws, r, pl.ds(cc, L)] = zero16

    @pl.when(s < NZ)
    def _():
        pltpu.sync_copy(z128_hbm.at[pl.ds(s * rps, rps)],
                        acc_sh.at[pl.ds(s * rps, rps)])
    plsc.subcore_barrier()

    ngr, ngc = col_hbm.shape[1], col_hbm.shape[2]
    ebase = wid * (ngr * ngc * kk)

    def wait_e():
        pltpu.make_async_copy(ea_hbm.at[pl.ds(0, kk)], ea_v.at[0], sem_e).wait()

    def wait_s():
        pltpu.make_async_copy(z128_hbm.at[pl.ds(0, kk)], w_v.at[0],
                              sem_s).wait()

    def group(g, _):
        pltpu.sync_copy(col_hbm.at[wid, g], cidx_v)
        gbase = ebase + g * ngc * kk
        pltpu.async_copy(ea_hbm.at[pl.ds(gbase, kk)], ea_v.at[0], sem_e)
        pltpu.async_copy(ea_hbm.at[pl.ds(gbase + kk, kk)], ea_v.at[1], sem_e)

        def chunk(ch, es):
            gs = es[1]
            ws = es[0]
            wait_e()

            @plsc.parallel_loop(0, kk, unroll=8)
            def _(r):
                w_v[ws, r, pl.ds(0, de)] = ea_v[gs, r, pl.ds(0, de)]

            @pl.when(ch >= 1)
            def _():
                wait_s()
            pltpu.async_copy(w_v.at[ws], acc_sh.at[cidx_v.at[ch]], sem_s,
                             add=True)

            @pl.when(ch + 2 < ngc)
            def _():
                pltpu.async_copy(
                    ea_hbm.at[pl.ds(gbase + (ch + 2) * kk, kk)],
                    ea_v.at[gs], sem_e)
            ws2 = jnp.where(ws == 2, 0, ws + 1)
            return (ws2, 1 - gs)

        lax.fori_loop(0, ngc, chunk, (0, 0))
        wait_s()
        return 0

    lax.fori_loop(0, ngr, group, 0)
    plsc.subcore_barrier()

    @pl.when(s < NZ)
    def _():
        pltpu.sync_copy(acc_sh.at[pl.ds(s * rps, rps)],
                        out_hbm.at[pl.ds(c * n + s * rps, rps)])


def _p5_body(n, d, kk, row_hbm, col_hbm, ew_hbm, gt_hbm, z128_hbm,
             out_hbm, ridx_v, cidx_v, ebuf, gbuf, acc_sh,
             sem_e, sem_g):
    c = lax.axis_index("c")
    s = lax.axis_index("s")
    wid = s * NC + c
    rps = n // NZ

    @pl.when(s < NZ)
    def _():
        pltpu.sync_copy(z128_hbm.at[pl.ds(s * rps, rps)],
                        acc_sh.at[pl.ds(s * rps, rps)])
    plsc.subcore_barrier()

    ngr, ngc = col_hbm.shape[1], col_hbm.shape[2]
    ebase = wid * (ngr * ngc * kk)

    def wait_e():
        pltpu.make_async_copy(ew_hbm.at[pl.ds(0, kk)], ebuf.at[0],
                              sem_e).wait()

    def wait_g():
        pltpu.make_async_copy(gt_hbm.at[pl.ds(0, kk)], gbuf.at[0],
                              sem_g).wait()

    def group(g, _):
        pltpu.sync_copy(row_hbm.at[wid, g], ridx_v)
        pltpu.sync_copy(col_hbm.at[wid, g], cidx_v)
        gbase = ebase + g * ngc * kk

        def fire(ch, es, gs):
            pltpu.async_copy(ew_hbm.at[pl.ds(gbase + ch * kk, kk)],
                             ebuf.at[es], sem_e)
            pltpu.async_copy(gt_hbm.at[ridx_v.at[ch]], gbuf.at[gs], sem_g)

        fire(0, 0, 0)
        fire(1, 1, 1)

        def chunk(ch, carry):
            es, gs = carry
            wait_e()
            wait_g()

            @plsc.parallel_loop(0, kk, unroll=4)
            def _(r):
                nr = gbuf[gs, r, pl.ds(d, L)]
                for cc in range(d // L):
                    sl = pl.ds(cc * L, L)
                    ebuf[es, r, sl] = jnp.maximum(
                        gbuf[gs, r, sl] + ebuf[es, r, sl] * nr, 0.0)

            pltpu.sync_copy(ebuf.at[es], acc_sh.at[cidx_v.at[ch]],
                            add=True)

            @pl.when(ch + 2 < ngc)
            def _():
                fire(ch + 2, es, gs)
            return (1 - es, 1 - gs)

        lax.fori_loop(0, ngc, chunk, (0, 0))
        return 0

    lax.fori_loop(0, ngr, group, 0)
    plsc.subcore_barrier()

    @pl.when(s < NZ)
    def _():
        pltpu.sync_copy(acc_sh.at[pl.ds(s * rps, rps)],
                        out_hbm.at[pl.ds(c * n + s * rps, rps)])


def _p2_body(x_ref, p0_ref, p1_ref, wmx_ref, wme_ref, wmb_ref,
             gt_ref, selfc_ref, normb_ref):
    de = wme_ref.shape[0]
    cnt = p0_ref[:, de:de + 1] + p1_ref[:, de:de + 1]
    loop_sum = p0_ref[:, :de] + p1_ref[:, :de]
    deg = cnt + 1.0
    norm = lax.rsqrt(deg)
    la = loop_sum / jnp.maximum(cnt, 1.0)
    xwb = (jnp.dot(x_ref[...], wmx_ref[...], preferred_element_type=F32)
           + wmb_ref[...])
    selfc = jnp.maximum(
        xwb + jnp.dot(la, wme_ref[...], preferred_element_type=F32), 0.0) / deg
    nb = jnp.broadcast_to(norm, normb_ref.shape)
    gt_ref[...] = jnp.concatenate([xwb * norm, nb], axis=1)
    selfc_ref[...] = selfc
    normb_ref[...] = nb


def _p4_body(ea_ref, wme_ref, out_ref):
    out_ref[...] = jnp.dot(ea_ref[...], wme_ref[...],
                           preferred_element_type=F32)


def _p6_body(x_ref, a0_ref, a1_ref, selfc_ref, normb_ref, batch_ref,
             wax_ref, waa_ref, wab_ref, sums_ref, cnts_ref):
    i = pl.program_id(0)

    @pl.when(i == 0)
    def _():
        sums_ref[...] = jnp.zeros_like(sums_ref)
        cnts_ref[...] = jnp.zeros_like(cnts_ref)

    aggr = (a0_ref[...] + a1_ref[...]) * normb_ref[...] + selfc_ref[...]
    h = jnp.maximum(
        jnp.dot(x_ref[...], wax_ref[...], preferred_element_type=F32)
        + jnp.dot(aggr, waa_ref[...], preferred_element_type=F32)
        + wab_ref[...], 0.0)
    oh = (batch_ref[...] ==
          lax.broadcasted_iota(jnp.int32, (1, G), 1)).astype(F32)
    sums_ref[...] += lax.dot_general(
        oh, h, (((0,), (0,)), ((), ())), preferred_element_type=F32)
    cnts_ref[...] += jnp.broadcast_to(
        jnp.sum(oh, axis=0)[:, None], cnts_ref.shape)


def _d3_body(sums_ref, cnts_ref, bng_ref, bnb_ref, p1t_ref, p1b_ref,
             p2t_ref, p2b_ref, out_ref):
    cnt = jnp.maximum(cnts_ref[:, :1], 1.0)
    g = sums_ref[...] / cnt
    g = g * (bng_ref[...] / jnp.sqrt(1.0 + EPS)) + bnb_ref[...]
    r = jnp.maximum(
        jnp.dot(g, p1t_ref[...], preferred_element_type=F32) + p1b_ref[...],
        0.0)
    out_ref[...] = (jnp.dot(r, p2t_ref[...], preferred_element_type=F32)
                    + p2b_ref[...])


def kernel(x, edge_index, edge_attr, batch, Wm_w, Wm_b, Wa_w, Wa_b,
           bn_gamma, bn_beta, P1_w, P1_b, P2_w, P2_b):
    n, d = x.shape
    e = edge_index.shape[1]
    de = edge_attr.shape[1]
    h = Wa_w.shape[0]
    p = P1_w.shape[0]
    t = P2_w.shape[0]

    et = e // NW
    k1, k5 = 40, 40
    nch1, nch5 = et // k1, et // k5
    ngc1 = 25 if nch1 % 25 == 0 else nch1
    ngc5 = 25 if nch5 % 25 == 0 else nch5
    ngr1, ngr5 = nch1 // ngc1, nch5 // ngc5
    nb = 1000 if n % 1000 == 0 else n
    eb = 2000 if e % 2000 == 0 else e

    col4a = edge_index[1].reshape(NW, ngr1, ngc1, k1)
    row4b = edge_index[0].reshape(NW, ngr5, ngc5, k5)
    col4b = edge_index[1].reshape(NW, ngr5, ngc5, k5)
    wmxT = Wm_w[:, :d].T
    wmeT = Wm_w[:, d:].T
    wmb2 = Wm_b.reshape(1, d)
    waxT = Wa_w[:, :d].T
    waaT = Wa_w[:, d:].T
    wab2 = Wa_b.reshape(1, h)
    batch2 = batch.reshape(n, 1)
    z128 = jnp.zeros((n, 128), F32)

    mesh = _sc_mesh()

    parts1 = pl.kernel(
        functools.partial(_p1_body, n, k1),
        out_type=jax.ShapeDtypeStruct((NC * n, 128), F32),
        mesh=mesh,
        scratch_types=[
            pltpu.VMEM((ngc1, k1), jnp.int32),
            pltpu.VMEM((2, k1, de), F32),
            pltpu.VMEM((3, k1, 128), F32),
            pltpu.VMEM_SHARED((n, 128), F32),
            pltpu.SemaphoreType.DMA,
            pltpu.SemaphoreType.DMA,
        ],
    )(col4a, edge_attr, z128)

    gt, selfc, normb = pl.pallas_call(
        _p2_body,
        grid=(n // nb,),
        in_specs=[
            pl.BlockSpec((nb, d), lambda i: (i, 0)),
            pl.BlockSpec((nb, 128), lambda i: (i, 0)),
            pl.BlockSpec((nb, 128), lambda i, _m=n // nb: (i + _m, 0)),
            pl.BlockSpec((d, d), lambda i: (0, 0)),
            pl.BlockSpec((de, d), lambda i: (0, 0)),
            pl.BlockSpec((1, d), lambda i: (0, 0)),
        ],
        out_specs=[
            pl.BlockSpec((nb, d + 128), lambda i: (i, 0)),
            pl.BlockSpec((nb, d), lambda i: (i, 0)),
            pl.BlockSpec((nb, 128), lambda i: (i, 0)),
        ],
        out_shape=[
            jax.ShapeDtypeStruct((n, d + 128), F32),
            jax.ShapeDtypeStruct((n, d), F32),
            jax.ShapeDtypeStruct((n, 128), F32),
        ],
    )(x, parts1, parts1, wmxT, wmeT, wmb2)

    ew = pl.pallas_call(
        _p4_body,
        grid=(e // eb,),
        in_specs=[
            pl.BlockSpec((eb, de), lambda i: (i, 0)),
            pl.BlockSpec((de, d), lambda i: (0, 0)),
        ],
        out_specs=pl.BlockSpec((eb, d), lambda i: (i, 0)),
        out_shape=jax.ShapeDtypeStruct((e, d), F32),
    )(edge_attr, wmeT)

    aggr_parts = pl.kernel(
        functools.partial(_p5_body, n, d, k5),
        out_type=jax.ShapeDtypeStruct((NC * n, d), F32),
        mesh=mesh,
        scratch_types=[
            pltpu.VMEM((ngc5, k5), jnp.int32),
            pltpu.VMEM((ngc5, k5), jnp.int32),
            pltpu.VMEM((2, k5, d), F32),
            pltpu.VMEM((2, k5, d + 128), F32),
            pltpu.VMEM_SHARED((n, d), F32),
            pltpu.SemaphoreType.DMA,
            pltpu.SemaphoreType.DMA,
        ],
    )(row4b, col4b, ew, gt, z128)

    sums, cnts = pl.pallas_call(
        _p6_body,
        grid=(n // nb,),
        in_specs=[
            pl.BlockSpec((nb, d), lambda i: (i, 0)),
            pl.BlockSpec((nb, d), lambda i: (i, 0)),
            pl.BlockSpec((nb, d), lambda i, _m=n // nb: (i + _m, 0)),
            pl.BlockSpec((nb, d), lambda i: (i, 0)),
            pl.BlockSpec((nb, 128), lambda i: (i, 0)),
            pl.BlockSpec((nb, 1), lambda i: (i, 0)),
            pl.BlockSpec((d, h), lambda i: (0, 0)),
            pl.BlockSpec((d, h), lambda i: (0, 0)),
            pl.BlockSpec((1, h), lambda i: (0, 0)),
        ],
        out_specs=[
            pl.BlockSpec((G, h), lambda i: (0, 0)),
            pl.BlockSpec((G, 128), lambda i: (0, 0)),
        ],
        out_shape=[
            jax.ShapeDtypeStruct((G, h), F32),
            jax.ShapeDtypeStruct((G, 128), F32),
        ],
    )(x, aggr_parts, aggr_parts, selfc, normb, batch2, waxT, waaT, wab2)

    out = pl.pallas_call(
        _d3_body,
        in_specs=[
            pl.BlockSpec((G, h), lambda: (0, 0)),
            pl.BlockSpec((G, 128), lambda: (0, 0)),
            pl.BlockSpec((1, h), lambda: (0, 0)),
            pl.BlockSpec((1, h), lambda: (0, 0)),
            pl.BlockSpec((h, p), lambda: (0, 0)),
            pl.BlockSpec((1, p), lambda: (0, 0)),
            pl.BlockSpec((p, t), lambda: (0, 0)),
            pl.BlockSpec((1, t), lambda: (0, 0)),
        ],
        out_specs=pl.BlockSpec((G, t), lambda: (0, 0)),
        out_shape=jax.ShapeDtypeStruct((G, t), F32),
    )(sums, cnts, bn_gamma.reshape(1, h), bn_beta.reshape(1, h),
      P1_w.T, P1_b.reshape(1, p), P2_w.T, P2_b.reshape(1, t))

    return out
